# baseline (speedup 1.0000x reference)
import functools

import jax
import jax.numpy as jnp
from jax import lax
from jax.experimental import pallas as pl
from jax.experimental.pallas import tpu as pltpu
from jax.experimental.pallas import tpu_sc as plsc

_LANES = 16
_CHUNK = 128


def kernel(inputs, token_table, pos_table):
    B, S = inputs.shape
    V, D = token_table.shape
    N = B * S
    NW = 32
    per_w = N // NW
    K = per_w // _CHUNK
    assert N % NW == 0 and per_w % _CHUNK == 0 and S % per_w == 0
    assert D % _LANES == 0

    idx3 = inputs.reshape(NW, K, _CHUNK).astype(jnp.int32)

    mesh = plsc.VectorSubcoreMesh(core_axis_name="c", subcore_axis_name="s")

    @functools.partial(
        pl.kernel,
        mesh=mesh,
        out_type=jax.ShapeDtypeStruct((N, D), jnp.float32),
        scratch_types=[
            pltpu.VMEM((K, _CHUNK), jnp.int32),
            pltpu.VMEM((K, _CHUNK, D), jnp.float32),
            pltpu.VMEM((K, _CHUNK, D), jnp.float32),
            pltpu.SemaphoreType.DMA,
        ],
        compiler_params=pltpu.CompilerParams(use_tc_tiling_on_sc=False),
    )
    def emb(idx_hbm, tok_hbm, pos_hbm, out_hbm, idx_v, rows_v, pos_v, sem):
        wid = lax.axis_index("s") * 2 + lax.axis_index("c")
        base = wid * per_w
        pos_base = base % S

        pltpu.sync_copy(idx_hbm.at[wid], idx_v)
        copies = []
        for j in range(K):
            copies.append(
                pltpu.async_copy(tok_hbm.at[idx_v.at[j]], rows_v.at[j], sem)
            )
            copies.append(
                pltpu.async_copy(
                    pos_hbm.at[pl.ds(pos_base + j * _CHUNK, _CHUNK)],
                    pos_v.at[j],
                    sem,
                )
            )
        for c in copies:
            c.wait()

        def add_row(r, _):
            for j in range(K):
                for v in range(D // _LANES):
                    sl = pl.ds(v * _LANES, _LANES)
                    rows_v[j, r, sl] = rows_v[j, r, sl] + pos_v[j, r, sl]
            return 0

        lax.fori_loop(0, _CHUNK, add_row, 0)

        for j in range(K):
            pltpu.sync_copy(
                rows_v.at[j], out_hbm.at[pl.ds(base + j * _CHUNK, _CHUNK)]
            )

    out = emb(idx3, token_table, pos_table.astype(jnp.float32))
    return out.reshape(B, S, D)
